# baseline (speedup 1.0000x reference)
import functools

import jax
import jax.numpy as jnp
from jax import lax
from jax.experimental import pallas as pl
from jax.experimental.pallas import tpu as pltpu
from jax.experimental.pallas import tpu_sc as plsc

N_TILE = 4096


def _tc_body(w_ref, x_ref, b_ref, vals_ref, pmax_ref, pidx_ref, *, n, t_tiles):
    t = pl.program_id(0)
    lhs = jnp.concatenate([w_ref[...], b_ref[...]], axis=0)
    vals = lax.dot_general(
        lhs, x_ref[...],
        dimension_numbers=(((0,), (1,)), ((), ())),
        preferred_element_type=jnp.float32)
    vals_ref[...] = vals
    iota = lax.broadcasted_iota(jnp.int32, vals.shape, 0)

    def epilogue(v):
        col_max = jnp.max(v, axis=0, keepdims=True)
        lidx = jnp.min(jnp.where(v == col_max, iota, N_TILE),
                       axis=0, keepdims=True)
        pmax_ref[...] = col_max.reshape(1, 1, v.shape[1])
        pidx_ref[...] = (lidx + t * N_TILE).reshape(1, 1, v.shape[1])

    @pl.when(t != t_tiles - 1)
    def _():
        epilogue(vals)

    @pl.when(t == t_tiles - 1)
    def _():
        epilogue(jnp.where(iota < n - t * N_TILE, vals, -jnp.inf))


def _sc_merge_body(pmax_hbm, pidx_hbm, out_hbm, vmax, vidx, vout, *, t_tiles):
    wid = lax.axis_index("s")

    @pl.when(wid < 8)
    def _():
        base = wid * 16
        pltpu.sync_copy(pmax_hbm, vmax)
        pltpu.sync_copy(pidx_hbm, vidx)
        m = vmax[0, 0, pl.ds(base, 16)]
        a = vidx[0, 0, pl.ds(base, 16)]
        for t in range(1, t_tiles):
            v = vmax[t, 0, pl.ds(base, 16)]
            i = vidx[t, 0, pl.ds(base, 16)]
            upd = v > m
            m = jnp.where(upd, v, m)
            a = jnp.where(upd, i, a)
        vout[...] = a
        pltpu.sync_copy(vout, out_hbm.at[pl.ds(base, 16)])


def kernel(x, W, b):
    bsz, d = x.shape
    n = W.shape[1]
    t_tiles = pl.cdiv(n, N_TILE)

    vals_t, pmax, pidx = pl.pallas_call(
        functools.partial(_tc_body, n=n, t_tiles=t_tiles),
        grid=(t_tiles,),
        in_specs=[
            pl.BlockSpec((d, N_TILE), lambda t: (0, t)),
            pl.BlockSpec((bsz, d + 1), lambda t: (0, 0)),
            pl.BlockSpec((1, N_TILE), lambda t: (0, t)),
        ],
        out_specs=[
            pl.BlockSpec((N_TILE, bsz), lambda t: (t, 0)),
            pl.BlockSpec((1, 1, bsz), lambda t: (t, 0, 0)),
            pl.BlockSpec((1, 1, bsz), lambda t: (t, 0, 0)),
        ],
        out_shape=[
            jax.ShapeDtypeStruct((n, bsz), jnp.float32),
            jax.ShapeDtypeStruct((t_tiles, 1, bsz), jnp.float32),
            jax.ShapeDtypeStruct((t_tiles, 1, bsz), jnp.int32),
        ],
        compiler_params=pltpu.CompilerParams(
            dimension_semantics=("parallel",)),
    )(W, jnp.concatenate([x, jnp.ones((bsz, 1), x.dtype)], axis=1),
      b.reshape(1, n))

    mesh = plsc.VectorSubcoreMesh(core_axis_name="c", subcore_axis_name="s",
                                  num_cores=1)
    sc_merge = functools.partial(
        pl.kernel,
        mesh=mesh,
        out_type=jax.ShapeDtypeStruct((bsz,), jnp.int32),
        scratch_types=[
            pltpu.VMEM((t_tiles, 1, bsz), jnp.float32),
            pltpu.VMEM((t_tiles, 1, bsz), jnp.int32),
            pltpu.VMEM((16,), jnp.int32),
        ],
    )(functools.partial(_sc_merge_body, t_tiles=t_tiles))

    actions = sc_merge(pmax, pidx)
    return (actions.astype(jnp.int64), vals_t.T)


# speedup vs baseline: 1.0897x; 1.0897x over previous
import functools

import jax
import jax.numpy as jnp
from jax import lax
from jax.experimental import pallas as pl
from jax.experimental.pallas import tpu as pltpu
from jax.experimental.pallas import tpu_sc as plsc

N_TILE = 16384


def _tc_body(w_ref, x_ref, b_ref, vals_ref, pmax_ref, pidx_ref, *, n, t_tiles):
    t = pl.program_id(0)
    lhs = jnp.concatenate([w_ref[...], b_ref[...]], axis=0)
    vals = lax.dot_general(
        lhs, x_ref[...],
        dimension_numbers=(((0,), (1,)), ((), ())),
        preferred_element_type=jnp.float32)
    vals_ref[...] = vals
    iota = lax.broadcasted_iota(jnp.int32, vals.shape, 0)

    def epilogue(v):
        col_max = jnp.max(v, axis=0, keepdims=True)
        lidx = jnp.min(jnp.where(v == col_max, iota, N_TILE),
                       axis=0, keepdims=True)
        pmax_ref[...] = col_max.reshape(1, 1, v.shape[1])
        pidx_ref[...] = (lidx + t * N_TILE).reshape(1, 1, v.shape[1])

    @pl.when(t != t_tiles - 1)
    def _():
        epilogue(vals)

    @pl.when(t == t_tiles - 1)
    def _():
        epilogue(jnp.where(iota < n - t * N_TILE, vals, -jnp.inf))


def _sc_merge_body(pmax_hbm, pidx_hbm, out_hbm, vmax, vidx, vout, *, t_tiles):
    wid = lax.axis_index("s")

    @pl.when(wid < 8)
    def _():
        base = wid * 16
        pltpu.sync_copy(pmax_hbm, vmax)
        pltpu.sync_copy(pidx_hbm, vidx)
        m = vmax[0, 0, pl.ds(base, 16)]
        a = vidx[0, 0, pl.ds(base, 16)]
        for t in range(1, t_tiles):
            v = vmax[t, 0, pl.ds(base, 16)]
            i = vidx[t, 0, pl.ds(base, 16)]
            upd = v > m
            m = jnp.where(upd, v, m)
            a = jnp.where(upd, i, a)
        vout[...] = a
        pltpu.sync_copy(vout, out_hbm.at[pl.ds(base, 16)])


def kernel(x, W, b):
    bsz, d = x.shape
    n = W.shape[1]
    t_tiles = pl.cdiv(n, N_TILE)

    vals_t, pmax, pidx = pl.pallas_call(
        functools.partial(_tc_body, n=n, t_tiles=t_tiles),
        grid=(t_tiles,),
        in_specs=[
            pl.BlockSpec((d, N_TILE), lambda t: (0, t)),
            pl.BlockSpec((bsz, d + 1), lambda t: (0, 0)),
            pl.BlockSpec((1, N_TILE), lambda t: (0, t)),
        ],
        out_specs=[
            pl.BlockSpec((N_TILE, bsz), lambda t: (t, 0)),
            pl.BlockSpec((1, 1, bsz), lambda t: (t, 0, 0)),
            pl.BlockSpec((1, 1, bsz), lambda t: (t, 0, 0)),
        ],
        out_shape=[
            jax.ShapeDtypeStruct((n, bsz), jnp.float32),
            jax.ShapeDtypeStruct((t_tiles, 1, bsz), jnp.float32),
            jax.ShapeDtypeStruct((t_tiles, 1, bsz), jnp.int32),
        ],
        compiler_params=pltpu.CompilerParams(
            dimension_semantics=("parallel",)),
    )(W, jnp.concatenate([x, jnp.ones((bsz, 1), x.dtype)], axis=1),
      b.reshape(1, n))

    mesh = plsc.VectorSubcoreMesh(core_axis_name="c", subcore_axis_name="s",
                                  num_cores=1)
    sc_merge = functools.partial(
        pl.kernel,
        mesh=mesh,
        out_type=jax.ShapeDtypeStruct((bsz,), jnp.int32),
        scratch_types=[
            pltpu.VMEM((t_tiles, 1, bsz), jnp.float32),
            pltpu.VMEM((t_tiles, 1, bsz), jnp.int32),
            pltpu.VMEM((16,), jnp.int32),
        ],
    )(functools.partial(_sc_merge_body, t_tiles=t_tiles))

    actions = sc_merge(pmax, pidx)
    return (actions.astype(jnp.int64), vals_t.T)


# speedup vs baseline: 1.0980x; 1.0077x over previous
import functools

import jax
import jax.numpy as jnp
from jax import lax
from jax.experimental import pallas as pl
from jax.experimental.pallas import tpu as pltpu
from jax.experimental.pallas import tpu_sc as plsc

N_TILE = 8192


def _tc_body(w_ref, x_ref, b_ref, vals_ref, pmax_ref, pidx_ref, *, n, t_tiles):
    t = pl.program_id(0)
    lhs = jnp.concatenate([w_ref[...], b_ref[...]], axis=0)
    vals = lax.dot_general(
        lhs, x_ref[...],
        dimension_numbers=(((0,), (1,)), ((), ())),
        preferred_element_type=jnp.float32)
    vals_ref[...] = vals
    iota = lax.broadcasted_iota(jnp.int32, vals.shape, 0)

    def epilogue(v):
        col_max = jnp.max(v, axis=0, keepdims=True)
        lidx = jnp.min(jnp.where(v == col_max, iota, N_TILE),
                       axis=0, keepdims=True)
        pmax_ref[...] = col_max.reshape(1, 1, v.shape[1])
        pidx_ref[...] = (lidx + t * N_TILE).reshape(1, 1, v.shape[1])

    @pl.when(t != t_tiles - 1)
    def _():
        epilogue(vals)

    @pl.when(t == t_tiles - 1)
    def _():
        epilogue(jnp.where(iota < n - t * N_TILE, vals, -jnp.inf))


def _sc_merge_body(pmax_hbm, pidx_hbm, out_hbm, vmax, vidx, vout, *, t_tiles):
    wid = lax.axis_index("s")

    @pl.when(wid < 8)
    def _():
        base = wid * 16
        pltpu.sync_copy(pmax_hbm, vmax)
        pltpu.sync_copy(pidx_hbm, vidx)
        m = vmax[0, 0, pl.ds(base, 16)]
        a = vidx[0, 0, pl.ds(base, 16)]
        for t in range(1, t_tiles):
            v = vmax[t, 0, pl.ds(base, 16)]
            i = vidx[t, 0, pl.ds(base, 16)]
            upd = v > m
            m = jnp.where(upd, v, m)
            a = jnp.where(upd, i, a)
        vout[...] = a
        pltpu.sync_copy(vout, out_hbm.at[pl.ds(base, 16)])


def kernel(x, W, b):
    bsz, d = x.shape
    n = W.shape[1]
    t_tiles = pl.cdiv(n, N_TILE)

    vals_t, pmax, pidx = pl.pallas_call(
        functools.partial(_tc_body, n=n, t_tiles=t_tiles),
        grid=(t_tiles,),
        in_specs=[
            pl.BlockSpec((d, N_TILE), lambda t: (0, t)),
            pl.BlockSpec((bsz, d + 1), lambda t: (0, 0)),
            pl.BlockSpec((1, N_TILE), lambda t: (0, t)),
        ],
        out_specs=[
            pl.BlockSpec((N_TILE, bsz), lambda t: (t, 0)),
            pl.BlockSpec((1, 1, bsz), lambda t: (t, 0, 0)),
            pl.BlockSpec((1, 1, bsz), lambda t: (t, 0, 0)),
        ],
        out_shape=[
            jax.ShapeDtypeStruct((n, bsz), jnp.float32),
            jax.ShapeDtypeStruct((t_tiles, 1, bsz), jnp.float32),
            jax.ShapeDtypeStruct((t_tiles, 1, bsz), jnp.int32),
        ],
        compiler_params=pltpu.CompilerParams(
            dimension_semantics=("parallel",)),
    )(W, jnp.concatenate([x, jnp.ones((bsz, 1), x.dtype)], axis=1),
      b.reshape(1, n))

    mesh = plsc.VectorSubcoreMesh(core_axis_name="c", subcore_axis_name="s",
                                  num_cores=1)
    sc_merge = functools.partial(
        pl.kernel,
        mesh=mesh,
        out_type=jax.ShapeDtypeStruct((bsz,), jnp.int32),
        scratch_types=[
            pltpu.VMEM((t_tiles, 1, bsz), jnp.float32),
            pltpu.VMEM((t_tiles, 1, bsz), jnp.int32),
            pltpu.VMEM((16,), jnp.int32),
        ],
    )(functools.partial(_sc_merge_body, t_tiles=t_tiles))

    actions = sc_merge(pmax, pidx)
    return (actions.astype(jnp.int64), vals_t.T)


# speedup vs baseline: 1.0991x; 1.0010x over previous
import functools

import jax
import jax.numpy as jnp
from jax import lax
from jax.experimental import pallas as pl
from jax.experimental.pallas import tpu as pltpu
from jax.experimental.pallas import tpu_sc as plsc

N_TILE = 8192


def _tc_body(w_ref, x_ref, b_ref, vals_ref, pmax_ref, pidx_ref, *, n, t_tiles):
    t = pl.program_id(0)
    lhs = jnp.concatenate([w_ref[...], b_ref[...]], axis=0)
    rhs = jnp.concatenate(
        [x_ref[...], jnp.ones((x_ref.shape[0], 1), jnp.float32)], axis=1)
    vals = lax.dot_general(
        lhs, rhs,
        dimension_numbers=(((0,), (1,)), ((), ())),
        preferred_element_type=jnp.float32)
    vals_ref[...] = vals
    iota = lax.broadcasted_iota(jnp.int32, vals.shape, 0)

    def epilogue(v):
        col_max = jnp.max(v, axis=0, keepdims=True)
        lidx = jnp.min(jnp.where(v == col_max, iota, N_TILE),
                       axis=0, keepdims=True)
        pmax_ref[...] = col_max.reshape(1, 1, v.shape[1])
        pidx_ref[...] = (lidx + t * N_TILE).reshape(1, 1, v.shape[1])

    @pl.when(t != t_tiles - 1)
    def _():
        epilogue(vals)

    @pl.when(t == t_tiles - 1)
    def _():
        epilogue(jnp.where(iota < n - t * N_TILE, vals, -jnp.inf))


def _sc_merge_body(pmax_hbm, pidx_hbm, out_hbm, vmax, vidx, vout, *, t_tiles):
    wid = lax.axis_index("s")

    @pl.when(wid < 8)
    def _():
        base = wid * 16
        pltpu.sync_copy(pmax_hbm, vmax)
        pltpu.sync_copy(pidx_hbm, vidx)
        m = vmax[0, 0, pl.ds(base, 16)]
        a = vidx[0, 0, pl.ds(base, 16)]
        for t in range(1, t_tiles):
            v = vmax[t, 0, pl.ds(base, 16)]
            i = vidx[t, 0, pl.ds(base, 16)]
            upd = v > m
            m = jnp.where(upd, v, m)
            a = jnp.where(upd, i, a)
        vout[...] = a
        pltpu.sync_copy(vout, out_hbm.at[pl.ds(base, 16)])


def kernel(x, W, b):
    bsz, d = x.shape
    n = W.shape[1]
    t_tiles = pl.cdiv(n, N_TILE)

    vals_t, pmax, pidx = pl.pallas_call(
        functools.partial(_tc_body, n=n, t_tiles=t_tiles),
        grid=(t_tiles,),
        in_specs=[
            pl.BlockSpec((d, N_TILE), lambda t: (0, t)),
            pl.BlockSpec((bsz, d), lambda t: (0, 0)),
            pl.BlockSpec((1, N_TILE), lambda t: (0, t)),
        ],
        out_specs=[
            pl.BlockSpec((N_TILE, bsz), lambda t: (t, 0)),
            pl.BlockSpec((1, 1, bsz), lambda t: (t, 0, 0)),
            pl.BlockSpec((1, 1, bsz), lambda t: (t, 0, 0)),
        ],
        out_shape=[
            jax.ShapeDtypeStruct((n, bsz), jnp.float32),
            jax.ShapeDtypeStruct((t_tiles, 1, bsz), jnp.float32),
            jax.ShapeDtypeStruct((t_tiles, 1, bsz), jnp.int32),
        ],
        compiler_params=pltpu.CompilerParams(
            dimension_semantics=("parallel",)),
    )(W, x, b.reshape(1, n))

    mesh = plsc.VectorSubcoreMesh(core_axis_name="c", subcore_axis_name="s",
                                  num_cores=1)
    sc_merge = functools.partial(
        pl.kernel,
        mesh=mesh,
        out_type=jax.ShapeDtypeStruct((bsz,), jnp.int32),
        scratch_types=[
            pltpu.VMEM((t_tiles, 1, bsz), jnp.float32),
            pltpu.VMEM((t_tiles, 1, bsz), jnp.int32),
            pltpu.VMEM((16,), jnp.int32),
        ],
    )(functools.partial(_sc_merge_body, t_tiles=t_tiles))

    actions = sc_merge(pmax, pidx)
    return (actions.astype(jnp.int64), vals_t.T)


# speedup vs baseline: 1.1356x; 1.0332x over previous
import functools

import jax
import jax.numpy as jnp
from jax import lax
from jax.experimental import pallas as pl
from jax.experimental.pallas import tpu as pltpu
from jax.experimental.pallas import tpu_sc as plsc

N_TILE = 8192


def _tc_body(w_ref, x_ref, b_ref, vals_ref, pmax_ref, pidx_ref, *, n, t_tiles):
    t = pl.program_id(0)
    lhs = jnp.concatenate([w_ref[...], b_ref[...].reshape(1, -1)], axis=0)
    rhs = jnp.concatenate(
        [x_ref[...], jnp.ones((x_ref.shape[0], 1), jnp.float32)], axis=1)
    vals = lax.dot_general(
        lhs, rhs,
        dimension_numbers=(((0,), (1,)), ((), ())),
        preferred_element_type=jnp.float32)
    vals_ref[...] = vals
    iota = lax.broadcasted_iota(jnp.int32, vals.shape, 0)

    def epilogue(v):
        col_max = jnp.max(v, axis=0, keepdims=True)
        lidx = jnp.min(jnp.where(v == col_max, iota, N_TILE),
                       axis=0, keepdims=True)
        pmax_ref[...] = col_max.reshape(1, 1, v.shape[1])
        pidx_ref[...] = (lidx + t * N_TILE).reshape(1, 1, v.shape[1])

    @pl.when(t != t_tiles - 1)
    def _():
        epilogue(vals)

    @pl.when(t == t_tiles - 1)
    def _():
        epilogue(jnp.where(iota < n - t * N_TILE, vals, -jnp.inf))


def _sc_merge_body(pmax_hbm, pidx_hbm, out_hbm, vmax, vidx, vout, *, t_tiles):
    wid = lax.axis_index("s")

    @pl.when(wid < 8)
    def _():
        base = wid * 16
        pltpu.sync_copy(pmax_hbm, vmax)
        pltpu.sync_copy(pidx_hbm, vidx)
        m = vmax[0, 0, pl.ds(base, 16)]
        a = vidx[0, 0, pl.ds(base, 16)]
        for t in range(1, t_tiles):
            v = vmax[t, 0, pl.ds(base, 16)]
            i = vidx[t, 0, pl.ds(base, 16)]
            upd = v > m
            m = jnp.where(upd, v, m)
            a = jnp.where(upd, i, a)
        vout[...] = a
        pltpu.sync_copy(vout, out_hbm.at[pl.ds(base, 16)])


def kernel(x, W, b):
    bsz, d = x.shape
    n = W.shape[1]
    t_tiles = pl.cdiv(n, N_TILE)

    vals_t, pmax, pidx = pl.pallas_call(
        functools.partial(_tc_body, n=n, t_tiles=t_tiles),
        grid=(t_tiles,),
        in_specs=[
            pl.BlockSpec((d, N_TILE), lambda t: (0, t)),
            pl.BlockSpec((bsz, d), lambda t: (0, 0)),
            pl.BlockSpec((N_TILE,), lambda t: (t,)),
        ],
        out_specs=[
            pl.BlockSpec((N_TILE, bsz), lambda t: (t, 0)),
            pl.BlockSpec((1, 1, bsz), lambda t: (t, 0, 0)),
            pl.BlockSpec((1, 1, bsz), lambda t: (t, 0, 0)),
        ],
        out_shape=[
            jax.ShapeDtypeStruct((n, bsz), jnp.float32),
            jax.ShapeDtypeStruct((t_tiles, 1, bsz), jnp.float32),
            jax.ShapeDtypeStruct((t_tiles, 1, bsz), jnp.int32),
        ],
        compiler_params=pltpu.CompilerParams(
            dimension_semantics=("parallel",)),
    )(W, x, b)

    mesh = plsc.VectorSubcoreMesh(core_axis_name="c", subcore_axis_name="s",
                                  num_cores=1)
    sc_merge = functools.partial(
        pl.kernel,
        mesh=mesh,
        out_type=jax.ShapeDtypeStruct((bsz,), jnp.int32),
        scratch_types=[
            pltpu.VMEM((t_tiles, 1, bsz), jnp.float32),
            pltpu.VMEM((t_tiles, 1, bsz), jnp.int32),
            pltpu.VMEM((16,), jnp.int32),
        ],
    )(functools.partial(_sc_merge_body, t_tiles=t_tiles))

    actions = sc_merge(pmax, pidx)
    return (actions.astype(jnp.int64), vals_t.T)
